# speedup vs baseline: 4.2962x; 4.2962x over previous
import jax
import jax.numpy as jnp
from jax import lax
from jax.experimental import pallas as pl
from jax.experimental.pallas import tpu as pltpu
from jax.experimental.pallas import tpu_sc as plsc

_LANES = 16
_IDX_PER_STREAM = 128
_CHUNK = 512


def _make_embed_kernel(batch, grid, vocab, d):
    b_total = batch * grid
    mesh = plsc.VectorSubcoreMesh(core_axis_name="c", subcore_axis_name="s")
    nc, ns = mesh.num_cores, mesh.num_subcores
    nw = nc * ns
    assert b_total % (nw * _CHUNK) == 0
    b_per_w = b_total // nw
    n_chunks = b_per_w // _CHUNK
    n_sub = _CHUNK // _IDX_PER_STREAM
    idx_rows_per_w = b_per_w // _IDX_PER_STREAM

    def body(idx_hbm, table_hbm, pos_hbm, out_hbm, idx_v, rows_v, pos_v, sem):
        wid = lax.axis_index("s") * nc + lax.axis_index("c")
        pltpu.sync_copy(pos_hbm, pos_v)

        @pl.loop(0, n_chunks)
        def _chunk(k):
            row_off = wid * b_per_w + k * _CHUNK
            idx_row_off = wid * idx_rows_per_w + k * n_sub
            pltpu.sync_copy(idx_hbm.at[pl.ds(idx_row_off, n_sub)], idx_v)
            copies = [
                pltpu.async_copy(
                    table_hbm.at[idx_v.at[t]],
                    rows_v.at[pl.ds(t * _IDX_PER_STREAM, _IDX_PER_STREAM)],
                    sem,
                )
                for t in range(n_sub)
            ]
            for c in copies:
                c.wait()

            @pl.loop(0, _CHUNK // grid)
            def _grp(g):
                r0 = g * grid
                for j in range(grid):
                    for v in range(d // _LANES):
                        sl = pl.ds(v * _LANES, _LANES)
                        rows_v[r0 + j, sl] = rows_v[r0 + j, sl] + pos_v[j, sl]

            pltpu.sync_copy(rows_v, out_hbm.at[pl.ds(row_off, _CHUNK)])

    return pl.kernel(
        body,
        out_type=jax.ShapeDtypeStruct((b_total, d), jnp.float32),
        mesh=mesh,
        scratch_types=[
            pltpu.VMEM((n_sub, _IDX_PER_STREAM), jnp.int32),
            pltpu.VMEM((_CHUNK, d), jnp.float32),
            pltpu.VMEM((grid, d), jnp.float32),
            pltpu.SemaphoreType.DMA,
        ],
    )


def kernel(tile_values, value_table, pos_table):
    batch, grid = tile_values.shape
    vocab, d = value_table.shape
    idx = tile_values.astype(jnp.int32).reshape(-1, _IDX_PER_STREAM)
    k = _make_embed_kernel(batch, grid, vocab, d)
    out = k(idx, value_table, pos_table)
    return out.reshape(batch, grid, d)


# speedup vs baseline: 7.4051x; 1.7237x over previous
import jax
import jax.numpy as jnp
from jax import lax
from jax.experimental import pallas as pl
from jax.experimental.pallas import tpu as pltpu
from jax.experimental.pallas import tpu_sc as plsc

_IDX_PER_STREAM = 128
_CHUNK = 512


def _make_embed_kernel(batch, grid, vocab, d):
    b_total = batch * grid
    mesh = plsc.VectorSubcoreMesh(core_axis_name="c", subcore_axis_name="s")
    nc, ns = mesh.num_cores, mesh.num_subcores
    nw = nc * ns
    assert b_total % (nw * _CHUNK) == 0
    b_per_w = b_total // nw
    n_chunks = b_per_w // _CHUNK
    n_sub = _CHUNK // _IDX_PER_STREAM
    idx_rows_per_w = b_per_w // _IDX_PER_STREAM

    def body(idx_hbm, table_hbm, patt_hbm, out_hbm, idx_v, rows_v, sem):
        wid = lax.axis_index("s") * nc + lax.axis_index("c")

        @pl.loop(0, n_chunks)
        def _chunk(k):
            row_off = wid * b_per_w + k * _CHUNK
            idx_row_off = wid * idx_rows_per_w + k * n_sub
            pltpu.sync_copy(patt_hbm, rows_v)
            pltpu.sync_copy(idx_hbm.at[pl.ds(idx_row_off, n_sub)], idx_v)
            copies = [
                pltpu.async_copy(
                    table_hbm.at[idx_v.at[t]],
                    rows_v.at[pl.ds(t * _IDX_PER_STREAM, _IDX_PER_STREAM)],
                    sem,
                    add=True,
                )
                for t in range(n_sub)
            ]
            for c in copies:
                c.wait()
            pltpu.sync_copy(rows_v, out_hbm.at[pl.ds(row_off, _CHUNK)])

    return pl.kernel(
        body,
        out_type=jax.ShapeDtypeStruct((b_total, d), jnp.float32),
        mesh=mesh,
        scratch_types=[
            pltpu.VMEM((n_sub, _IDX_PER_STREAM), jnp.int32),
            pltpu.VMEM((_CHUNK, d), jnp.float32),
            pltpu.SemaphoreType.DMA,
        ],
    )


def kernel(tile_values, value_table, pos_table):
    batch, grid = tile_values.shape
    vocab, d = value_table.shape
    idx = tile_values.astype(jnp.int32).reshape(-1, _IDX_PER_STREAM)
    patt = jnp.tile(pos_table, (_CHUNK // grid, 1))
    k = _make_embed_kernel(batch, grid, vocab, d)
    out = k(idx, value_table, patt)
    return out.reshape(batch, grid, d)
